# baseline (speedup 1.0000x reference)
import jax
import jax.numpy as jnp
from jax.experimental import pallas as pl
from jax.experimental.pallas import tpu as pltpu
from jax.experimental.pallas import tpu_sc as plsc

EMBED_DIM = 128
WINDOW = 128


def _gather_sc(table, flat_ids):
    num_indices = flat_ids.shape[0]
    ids2d = flat_ids.reshape(1, num_indices)
    mesh = plsc.VectorSubcoreMesh(core_axis_name="c", subcore_axis_name="s")

    @pl.kernel(
        out_type=jax.ShapeDtypeStruct((num_indices, EMBED_DIM), table.dtype),
        mesh=mesh,
    )
    def gather_kernel(table_hbm, ids_hbm, out_hbm):
        def body(ids_vmem, out_vmem):
            pltpu.sync_copy(table_hbm.at[ids_vmem.at[0]], out_vmem)

        pltpu.emit_pipeline(
            body,
            grid=(num_indices // WINDOW,),
            in_specs=[pl.BlockSpec((1, WINDOW), index_map=lambda i: (0, i))],
            out_specs=[pl.BlockSpec((WINDOW, EMBED_DIM), index_map=lambda i: (i, 0))],
            core_axis_name=("c", "s"),
            dimension_semantics=(pltpu.PARALLEL,),
        )(ids_hbm, out_hbm)

    return gather_kernel(table, ids2d)


def kernel(token_ids, table):
    batch, seq = token_ids.shape
    flat = token_ids.reshape(batch * seq).astype(jnp.int32)
    out = _gather_sc(table, flat)
    return out.reshape(batch, seq, EMBED_DIM)


# speedup vs baseline: 1.2102x; 1.2102x over previous
import functools

import jax
import jax.numpy as jnp
from jax import lax
from jax.experimental import pallas as pl
from jax.experimental.pallas import tpu as pltpu
from jax.experimental.pallas import tpu_sc as plsc

EMBED_DIM = 128
NUM_CORES = 2
NUM_SUBCORES = 16
NUM_WORKERS = NUM_CORES * NUM_SUBCORES
CHUNK = 400


def _gather_sc(table, flat_ids):
    num_indices = flat_ids.shape[0]
    per_worker = num_indices // NUM_WORKERS
    nchunks = per_worker // CHUNK
    assert per_worker % CHUNK == 0
    mesh = plsc.VectorSubcoreMesh(core_axis_name="c", subcore_axis_name="s")

    @functools.partial(
        pl.kernel,
        mesh=mesh,
        out_type=jax.ShapeDtypeStruct((num_indices, EMBED_DIM), table.dtype),
        scratch_types=[
            pltpu.VMEM((per_worker,), jnp.int32),
            pltpu.VMEM((CHUNK, EMBED_DIM), jnp.float32),
            pltpu.VMEM((CHUNK, EMBED_DIM), jnp.float32),
            pltpu.SemaphoreType.DMA,
            pltpu.SemaphoreType.DMA,
            pltpu.SemaphoreType.DMA,
            pltpu.SemaphoreType.DMA,
        ],
    )
    def gather_kernel(table_hbm, ids_hbm, out_hbm, idx_v, buf0, buf1,
                      gsem0, gsem1, wsem0, wsem1):
        wid = lax.axis_index("s") * NUM_CORES + lax.axis_index("c")
        base = wid * per_worker
        pltpu.sync_copy(ids_hbm.at[pl.ds(base, per_worker)], idx_v)

        def start_gather(c, buf, sem):
            pltpu.async_copy(table_hbm.at[idx_v.at[pl.ds(c * CHUNK, CHUNK)]],
                             buf, sem)

        def wait_gather(c, buf, sem):
            pltpu.make_async_copy(
                table_hbm.at[idx_v.at[pl.ds(c * CHUNK, CHUNK)]], buf, sem
            ).wait()

        def start_write(c, buf, sem):
            pltpu.async_copy(buf, out_hbm.at[pl.ds(base + c * CHUNK, CHUNK)],
                             sem)

        def wait_write(c, buf, sem):
            pltpu.make_async_copy(
                buf, out_hbm.at[pl.ds(base + c * CHUNK, CHUNK)], sem
            ).wait()

        start_gather(0, buf0, gsem0)
        start_gather(1, buf1, gsem1)

        @pl.loop(0, nchunks, step=2)
        def _(g):
            wait_gather(g, buf0, gsem0)
            start_write(g, buf0, wsem0)
            wait_gather(g + 1, buf1, gsem1)
            start_write(g + 1, buf1, wsem1)

            @pl.when(g + 2 < nchunks)
            def _():
                wait_write(g, buf0, wsem0)
                start_gather(g + 2, buf0, gsem0)

            @pl.when(g + 3 < nchunks)
            def _():
                wait_write(g + 1, buf1, wsem1)
                start_gather(g + 3, buf1, gsem1)

        wait_write(nchunks - 2, buf0, wsem0)
        wait_write(nchunks - 1, buf1, wsem1)

    return gather_kernel(table, flat_ids)


def kernel(token_ids, table):
    batch, seq = token_ids.shape
    flat = token_ids.reshape(batch * seq).astype(jnp.int32)
    out = _gather_sc(table, flat)
    return out.reshape(batch, seq, EMBED_DIM)


# speedup vs baseline: 1.2314x; 1.0175x over previous
import jax
import jax.numpy as jnp
from jax.experimental import pallas as pl
from jax.experimental.pallas import tpu as pltpu
from jax.experimental.pallas import tpu_sc as plsc

EMBED_DIM = 128
WINDOW = 256


def _gather_sc(table, flat_ids):
    num_indices = flat_ids.shape[0]
    ids2d = flat_ids.reshape(1, num_indices)
    mesh = plsc.VectorSubcoreMesh(core_axis_name="c", subcore_axis_name="s")

    @pl.kernel(
        out_type=jax.ShapeDtypeStruct((num_indices, EMBED_DIM), table.dtype),
        mesh=mesh,
    )
    def gather_kernel(table_hbm, ids_hbm, out_hbm):
        def body(ids_vmem, out_vmem):
            pltpu.sync_copy(table_hbm.at[ids_vmem.at[0]], out_vmem)

        pltpu.emit_pipeline(
            body,
            grid=(num_indices // WINDOW,),
            in_specs=[pl.BlockSpec((1, WINDOW), index_map=lambda i: (0, i))],
            out_specs=[pl.BlockSpec((WINDOW, EMBED_DIM), index_map=lambda i: (i, 0))],
            core_axis_name=("c", "s"),
            dimension_semantics=(pltpu.PARALLEL,),
        )(ids_hbm, out_hbm)

    return gather_kernel(table, ids2d)


def kernel(token_ids, table):
    batch, seq = token_ids.shape
    flat = token_ids.reshape(batch * seq).astype(jnp.int32)
    out = _gather_sc(table, flat)
    return out.reshape(batch, seq, EMBED_DIM)
